# baseline (speedup 1.0000x reference)
import jax
import jax.numpy as jnp
from jax.experimental import pallas as pl
from jax.experimental.pallas import tpu as pltpu

N = 2048
BLK = 256
NBLK = N // BLK


def _body(src_ref, tgt_ref, out_ref, colmin_ref, rowsum_ref):
    i = pl.program_id(0)
    src = src_ref[...]
    tgt = tgt_ref[...]
    dot = jax.lax.dot_general(
        src, tgt, (((1,), (1,)), ((), ())),
        preferred_element_type=jnp.float32,
        precision=jax.lax.Precision.DEFAULT,
    )
    ss = jnp.sum(src * src, axis=1, keepdims=True)
    ones = jnp.ones((1, 3), dtype=jnp.float32)
    tt = jax.lax.dot_general(
        ones, tgt * tgt, (((1,), (1,)), ((), ())),
        preferred_element_type=jnp.float32,
        precision=jax.lax.Precision.HIGHEST,
    )
    dist = ss + tt - 2.0 * dot

    rs = jnp.sum(jnp.min(dist, axis=1))
    cm = jnp.min(dist, axis=0, keepdims=True)

    @pl.when(i == 0)
    def _():
        colmin_ref[...] = cm
        rowsum_ref[0] = rs

    @pl.when(i > 0)
    def _():
        colmin_ref[...] = jnp.minimum(colmin_ref[...], cm)
        rowsum_ref[0] = rowsum_ref[0] + rs

    @pl.when(i == NBLK - 1)
    def _():
        loss_s2t = rowsum_ref[0] / N
        loss_t2s = jnp.sum(colmin_ref[...]) / N
        out_ref[0, 0] = loss_s2t + 0.8 * loss_t2s


def kernel(source_cloud, target_cloud):
    out = pl.pallas_call(
        _body,
        grid=(NBLK,),
        in_specs=[
            pl.BlockSpec((BLK, 3), lambda i: (i, 0)),
            pl.BlockSpec((N, 3), lambda i: (0, 0)),
        ],
        out_specs=pl.BlockSpec(memory_space=pltpu.SMEM),
        out_shape=jax.ShapeDtypeStruct((1, 1), jnp.float32),
        scratch_shapes=[
            pltpu.VMEM((1, N), jnp.float32),
            pltpu.SMEM((1,), jnp.float32),
        ],
    )(source_cloud, target_cloud)
    return out[0, 0]


# speedup vs baseline: 1.7807x; 1.7807x over previous
import jax
import jax.numpy as jnp
from jax.experimental import pallas as pl
from jax.experimental.pallas import tpu as pltpu

N = 2048
BLK = 256
NBLK = N // BLK


def _body(src_ref, tgtT_ref, out_ref, colmin_ref, tt_ref, rowsum_ref):
    i = pl.program_id(0)
    src = src_ref[...]
    tgtT = tgtT_ref[...]

    @pl.when(i == 0)
    def _():
        tt_ref[...] = jnp.sum(tgtT * tgtT, axis=0, keepdims=True)

    dot = jax.lax.dot_general(
        src, tgtT, (((1,), (0,)), ((), ())),
        preferred_element_type=jnp.float32,
        precision=jax.lax.Precision.DEFAULT,
    )
    ss = jnp.sum(src * src, axis=1, keepdims=True)
    dist = (tt_ref[...] - 2.0 * dot) + ss

    rs = jnp.sum(jnp.min(dist, axis=1))
    cm = jnp.min(dist, axis=0, keepdims=True)

    @pl.when(i == 0)
    def _():
        colmin_ref[...] = cm
        rowsum_ref[0] = rs

    @pl.when(i > 0)
    def _():
        colmin_ref[...] = jnp.minimum(colmin_ref[...], cm)
        rowsum_ref[0] = rowsum_ref[0] + rs

    @pl.when(i == NBLK - 1)
    def _():
        loss_s2t = rowsum_ref[0] / N
        loss_t2s = jnp.sum(colmin_ref[...]) / N
        out_ref[0, 0] = loss_s2t + 0.8 * loss_t2s


def kernel(source_cloud, target_cloud):
    tgtT = target_cloud.T
    out = pl.pallas_call(
        _body,
        grid=(NBLK,),
        in_specs=[
            pl.BlockSpec((BLK, 3), lambda i: (i, 0)),
            pl.BlockSpec((3, N), lambda i: (0, 0)),
        ],
        out_specs=pl.BlockSpec(memory_space=pltpu.SMEM),
        out_shape=jax.ShapeDtypeStruct((1, 1), jnp.float32),
        scratch_shapes=[
            pltpu.VMEM((1, N), jnp.float32),
            pltpu.VMEM((1, N), jnp.float32),
            pltpu.SMEM((1,), jnp.float32),
        ],
    )(source_cloud, tgtT)
    return out[0, 0]
